# baseline (speedup 1.0000x reference)
import jax
import jax.numpy as jnp
from jax.experimental import pallas as pl
from jax.experimental.pallas import tpu as pltpu

_NUM_EXPERTS = 64
_TOPK = 8
_BT = 2048
_CH = 512


def _router_kernel(x_ref, w_ref, probs_ref, map_ref):
    x = x_ref[...]
    w = w_ref[...]
    logits = jax.lax.dot_general(
        w, x, (((1,), (1,)), ((), ())), preferred_element_type=jnp.float32
    )
    eidx = jax.lax.broadcasted_iota(
        jnp.int32, (_NUM_EXPERTS, _CH), 0
    ).astype(jnp.float32)
    neg = jnp.float32(-jnp.inf)
    for c in range(_BT // _CH):
        lg = logits[:, c * _CH : (c + 1) * _CH]
        work = lg
        m0 = None
        for r in range(_TOPK):
            m = jnp.max(work, axis=0, keepdims=True)
            if r == 0:
                m0 = m
            first = jnp.min(
                jnp.where(work == m, eidx, jnp.float32(_NUM_EXPERTS)),
                axis=0,
                keepdims=True,
            )
            work = jnp.where(eidx == first, neg, work)
        sel = work != lg
        e = jnp.where(sel, jnp.exp(lg - m0), jnp.float32(0.0))
        denom = jnp.sum(e, axis=0, keepdims=True)
        probs_t = e * (jnp.float32(1.0) / denom)
        p = probs_t.T
        probs_ref[pl.ds(c * _CH, _CH), :] = p
        map_ref[pl.ds(c * _CH, _CH), :] = p > jnp.float32(0.0)


@jax.jit
def kernel(input, W):
    num_tokens, d_model = input.shape
    grid = (num_tokens // _BT,)
    probs, rmap = pl.pallas_call(
        _router_kernel,
        grid=grid,
        in_specs=[
            pl.BlockSpec((_BT, d_model), lambda i: (i, 0)),
            pl.BlockSpec((_NUM_EXPERTS, d_model), lambda i: (0, 0)),
        ],
        out_specs=[
            pl.BlockSpec((_BT, _NUM_EXPERTS), lambda i: (i, 0)),
            pl.BlockSpec((_BT, _NUM_EXPERTS), lambda i: (i, 0)),
        ],
        out_shape=[
            jax.ShapeDtypeStruct((num_tokens, _NUM_EXPERTS), jnp.float32),
            jax.ShapeDtypeStruct((num_tokens, _NUM_EXPERTS), jnp.bool_),
        ],
        compiler_params=pltpu.CompilerParams(
            dimension_semantics=("parallel",),
        ),
    )(input, W)
    return probs, rmap
